# baseline (speedup 1.0000x reference)
import functools

import jax
import jax.numpy as jnp
from jax import lax
from jax.experimental import pallas as pl
from jax.experimental.pallas import tpu as pltpu
from jax.experimental.pallas import tpu_sc as plsc

R, C = 128, 32768
L = 16
SC_ROWS = 16
TC_ROWS = R - SC_ROWS
U = 4
SEG = 4096
NSEG = C // SEG
NOSEG = 2

_TC_BR = 16


def _sc_body(in_hbm, out_hbm, row_f, ob0, ob1, red16,
             is0, is1, is2, is3, os0, os1):
    wid = lax.axis_index("s")

    in_sems = (is0, is1, is2, is3)
    out_bufs = (ob0, ob1)
    out_sems = (os0, os1)

    neg_inf = jnp.full((L,), -jnp.inf, jnp.float32)
    one = jnp.full((L,), 1, jnp.int32)
    zero = jnp.full((L,), 0, jnp.int32)

    seg_chunks = SEG // (L * U)

    in_dma = []
    for j in range(NSEG):
        in_dma.append(pltpu.async_copy(
            in_hbm.at[wid, pl.ds(j * SEG, SEG)],
            row_f.at[pl.ds(j * SEG, SEG)],
            in_sems[j % 4]))

    accs = (neg_inf,) * U
    for j in range(NSEG):
        in_dma[j].wait()

        def max_body(k, a, j=j):
            base = j * SEG + k * (L * U)
            return tuple(
                jnp.maximum(x, row_f[pl.ds(base + jj * L, L)])
                for jj, x in enumerate(a)
            )

        accs = lax.fori_loop(0, seg_chunks, max_body, accs)

    acc = functools.reduce(jnp.maximum, accs)
    srt, _ = plsc.sort_key_val(acc, acc)
    red16[...] = srt
    mxv = plsc.load_gather(red16, [jnp.full((L,), L - 1, jnp.int32)])

    out_dma = [None] * NOSEG
    for j in range(NSEG):
        s = j % NOSEG
        ob = out_bufs[s]
        if out_dma[s] is not None:
            out_dma[s].wait()

        def cmp_body(k, carry, j=j, ob=ob):
            src = j * SEG + k * (L * U)
            dst = k * (L * U)
            for jj in range(U):
                v = row_f[pl.ds(src + jj * L, L)]
                ob[pl.ds(dst + jj * L, L)] = jnp.where(v == mxv, one, zero)
            return carry

        lax.fori_loop(0, seg_chunks, cmp_body, 0)
        out_dma[s] = pltpu.async_copy(
            ob, out_hbm.at[wid, pl.ds(j * SEG, SEG)], out_sems[s])

    for s in range(NOSEG):
        out_dma[s].wait()


def _sc_part(input):
    mesh = plsc.VectorSubcoreMesh(
        core_axis_name="c", subcore_axis_name="s", num_cores=1)
    k = pl.kernel(
        _sc_body,
        out_type=jax.ShapeDtypeStruct((SC_ROWS, C), jnp.int32),
        mesh=mesh,
        scratch_types=[
            pltpu.VMEM((C,), jnp.float32),
            pltpu.VMEM((SEG,), jnp.int32),
            pltpu.VMEM((SEG,), jnp.int32),
            pltpu.VMEM((L,), jnp.float32),
            pltpu.SemaphoreType.DMA,
            pltpu.SemaphoreType.DMA,
            pltpu.SemaphoreType.DMA,
            pltpu.SemaphoreType.DMA,
            pltpu.SemaphoreType.DMA,
            pltpu.SemaphoreType.DMA,
        ],
        compiler_params=pltpu.CompilerParams(
            needs_layout_passes=False, vmem_limit_bytes=1 << 20),
    )
    return k(input)


def _tc_body(x_ref, o_ref):
    x = x_ref[...]
    m3 = jnp.max(x.reshape(_TC_BR, C // 128, 128), axis=1)
    m = jnp.max(m3, axis=1, keepdims=True)
    o_ref[...] = (x == m).astype(jnp.int32)


def _tc_part(input):
    return pl.pallas_call(
        _tc_body,
        grid=(TC_ROWS // _TC_BR,),
        in_specs=[pl.BlockSpec((_TC_BR, C), lambda i: (i + SC_ROWS // _TC_BR, 0))],
        out_specs=pl.BlockSpec((_TC_BR, C), lambda i: (i + SC_ROWS // _TC_BR, 0)),
        out_shape=jax.ShapeDtypeStruct((R, C), jnp.int32),
    )(input)


def _patch_body(src_ref, _, o_ref):
    o_ref[...] = src_ref[...]


def _merge(tc_full, sc_out):
    return pl.pallas_call(
        _patch_body,
        grid=(1,),
        in_specs=[
            pl.BlockSpec((SC_ROWS, C), lambda i: (i, 0)),
            pl.BlockSpec(memory_space=pltpu.MemorySpace.HBM),
        ],
        out_specs=pl.BlockSpec((SC_ROWS, C), lambda i: (i, 0)),
        out_shape=jax.ShapeDtypeStruct((R, C), jnp.int32),
        input_output_aliases={1: 0},
    )(sc_out, tc_full)


def kernel(input):
    sc_out = _sc_part(input)
    tc_full = _tc_part(input)
    return _merge(tc_full, sc_out)


# speedup vs baseline: 1.0370x; 1.0370x over previous
import functools

import jax
import jax.numpy as jnp
from jax import lax
from jax.experimental import pallas as pl
from jax.experimental.pallas import tpu as pltpu
from jax.experimental.pallas import tpu_sc as plsc

R, C = 128, 32768
H = C // 2
L = 16
NC, NS = 2, 16
NW = NC * NS
ROWS_PER_W = R // NW
U = 8


def _body(in_hbm, out_hbm, in0, in1, out0, out1, red16,
          isem0, isem1, osem0, osem1):
    wid = lax.axis_index("s") * NC + lax.axis_index("c")
    row0 = wid * ROWS_PER_W

    in_bufs = (in0, in1)
    in_sems = (isem0, isem1)
    out_bufs = (out0, out1)
    out_sems = (osem0, osem1)

    neg_inf = jnp.full((L,), -jnp.inf, jnp.float32)
    one = jnp.full((L,), 1, jnp.int32)
    zero = jnp.full((L,), 0, jnp.int32)

    n_chunks = C // (L * U)
    n_chunks_h = H // (L * U)

    in_dma = [None, None]
    out_dma = [None, None]

    in_dma[0] = pltpu.async_copy(in_hbm.at[row0], in0, isem0)

    for i in range(ROWS_PER_W):
        buf = in_bufs[i % 2]
        in_dma[i % 2].wait()
        if i + 1 < ROWS_PER_W:
            nxt = (i + 1) % 2
            in_dma[nxt] = pltpu.async_copy(
                in_hbm.at[row0 + i + 1], in_bufs[nxt], in_sems[nxt])

        def max_body(k, accs):
            base = k * (L * U)
            return tuple(
                jnp.maximum(a, buf[pl.ds(base + j * L, L)])
                for j, a in enumerate(accs)
            )

        accs = lax.fori_loop(0, n_chunks, max_body, (neg_inf,) * U)
        acc = functools.reduce(jnp.maximum, accs)
        srt, _ = plsc.sort_key_val(acc, acc)
        red16[...] = srt
        mxv = plsc.load_gather(red16, [jnp.full((L,), L - 1, jnp.int32)])

        for h in range(2):
            ob = out_bufs[h]
            if out_dma[h] is not None:
                out_dma[h].wait()

            def cmp_body(k, carry):
                src = h * H + k * (L * U)
                dst = k * (L * U)
                for j in range(U):
                    v = buf[pl.ds(src + j * L, L)]
                    ob[pl.ds(dst + j * L, L)] = jnp.where(v == mxv, one, zero)
                return carry

            lax.fori_loop(0, n_chunks_h, cmp_body, 0)
            out_dma[h] = pltpu.async_copy(
                ob, out_hbm.at[row0 + i, pl.ds(h * H, H)], out_sems[h])

    out_dma[0].wait()
    out_dma[1].wait()


def kernel(input):
    mesh = plsc.VectorSubcoreMesh(core_axis_name="c", subcore_axis_name="s")
    k = pl.kernel(
        _body,
        out_type=jax.ShapeDtypeStruct((R, C), jnp.int32),
        mesh=mesh,
        scratch_types=[
            pltpu.VMEM((C,), jnp.float32),
            pltpu.VMEM((C,), jnp.float32),
            pltpu.VMEM((H,), jnp.int32),
            pltpu.VMEM((H,), jnp.int32),
            pltpu.VMEM((L,), jnp.float32),
            pltpu.SemaphoreType.DMA,
            pltpu.SemaphoreType.DMA,
            pltpu.SemaphoreType.DMA,
            pltpu.SemaphoreType.DMA,
        ],
        compiler_params=pltpu.CompilerParams(needs_layout_passes=False),
    )
    return k(input)
